# Initial kernel scaffold; baseline (speedup 1.0000x reference)
import jax
import jax.numpy as jnp
from jax.experimental import pallas as pl


def kernel(q, k, v, k_sparse):
    raise NotImplementedError("write your pallas kernel here")


# speedup vs baseline: 23.5718x; 23.5718x over previous
import math

import jax
import jax.numpy as jnp
from jax.experimental import pallas as pl
from jax.experimental.pallas import tpu as pltpu

_K = 32


def _topk_attn_kernel(q_ref, k_ref, v_ref, o_ref):
    qb = pl.program_id(1)
    QB = q_ref.shape[1]
    T = k_ref.shape[1]
    D = q_ref.shape[2]
    scale = 1.0 / math.sqrt(D)

    q = q_ref[0]
    k = k_ref[0]
    s = jax.lax.dot_general(
        q, k, (((1,), (1,)), ((), ())), preferred_element_type=jnp.float32
    ) * scale

    row = qb * QB + jax.lax.broadcasted_iota(jnp.int32, (QB, T), 0)
    col = jax.lax.broadcasted_iota(jnp.int32, (QB, T), 1)
    s = jnp.where(col > row, -jnp.inf, s)

    u = jax.lax.bitcast_convert_type(s, jnp.uint32)
    key = jnp.where(
        u >= jnp.uint32(0x80000000), ~u, u | jnp.uint32(0x80000000)
    )

    thr = jnp.zeros((QB, 1), jnp.uint32)
    for b in range(31, -1, -1):
        cand = thr | jnp.uint32(1 << b)
        cnt = jnp.sum((key >= cand).astype(jnp.int32), axis=1, keepdims=True)
        thr = jnp.where(cnt >= _K, cand, thr)

    m = jnp.max(s, axis=1, keepdims=True)
    p = jnp.where(key >= thr, jnp.exp(s - m), 0.0)
    denom = jnp.sum(p, axis=1, keepdims=True)
    o = jax.lax.dot_general(
        p, v_ref[0], (((1,), (0,)), ((), ())), preferred_element_type=jnp.float32
    )
    o_ref[0] = o / denom


def kernel(q, k, v, k_sparse):
    B, H, T, D = q.shape
    QB = 256
    BH = B * H
    q3 = q.reshape(BH, T, D)
    k3 = k.reshape(BH, T, D)
    v3 = v.reshape(BH, T, D)

    out = pl.pallas_call(
        _topk_attn_kernel,
        grid=(BH, T // QB),
        in_specs=[
            pl.BlockSpec((1, QB, D), lambda bh, qb: (bh, qb, 0)),
            pl.BlockSpec((1, T, D), lambda bh, qb: (bh, 0, 0)),
            pl.BlockSpec((1, T, D), lambda bh, qb: (bh, 0, 0)),
        ],
        out_specs=pl.BlockSpec((1, QB, D), lambda bh, qb: (bh, qb, 0)),
        out_shape=jax.ShapeDtypeStruct((BH, T, D), jnp.float32),
        compiler_params=pltpu.CompilerParams(
            dimension_semantics=("parallel", "arbitrary"),
        ),
    )(q3, k3, v3)
    return out.reshape(B, H, T, D)


# speedup vs baseline: 39.9943x; 1.6967x over previous
import math

import jax
import jax.numpy as jnp
from jax.experimental import pallas as pl
from jax.experimental.pallas import tpu as pltpu

_K = 32
_QB = 256


def _topk_attn_kernel(q_ref, k_ref, v_ref, o_ref, *, qi):
    QB = q_ref.shape[1]
    L = k_ref.shape[1]
    D = q_ref.shape[2]
    scale = 1.0 / math.sqrt(D)

    q = q_ref[0]
    k = k_ref[0]
    s = jax.lax.dot_general(
        q, k, (((1,), (1,)), ((), ())), preferred_element_type=jnp.float32
    ) * scale

    row = qi * QB + jax.lax.broadcasted_iota(jnp.int32, (QB, L), 0)
    col = jax.lax.broadcasted_iota(jnp.int32, (QB, L), 1)
    s = jnp.where(col > row, -jnp.inf, s)

    u = jax.lax.bitcast_convert_type(s, jnp.uint32)
    key = jnp.where(u >= jnp.uint32(0x80000000), ~u, u | jnp.uint32(0x80000000))

    thr = jnp.zeros((QB, 1), jnp.uint32)
    for b in range(31, 3, -1):
        cand = thr | jnp.uint32(1 << b)
        cnt = jnp.count_nonzero(key >= cand, axis=1, keepdims=True)
        thr = jnp.where(cnt >= _K, cand, thr)

    m = jnp.max(s, axis=1, keepdims=True)
    p = jnp.where(key >= thr, jnp.exp(s - m), 0.0)
    denom = jnp.sum(p, axis=1, keepdims=True)
    o = jax.lax.dot_general(
        p, v_ref[0], (((1,), (0,)), ((), ())), preferred_element_type=jnp.float32
    )
    o_ref[0] = o / denom


def kernel(q, k, v, k_sparse):
    import functools

    B, H, T, D = q.shape
    QB = _QB
    BH = B * H
    NQ = T // QB
    q3 = q.reshape(BH, T, D)
    k3 = k.reshape(BH, T, D)
    v3 = v.reshape(BH, T, D)

    outs = []
    for qi in range(NQ):
        L = (qi + 1) * QB
        out_qi = pl.pallas_call(
            functools.partial(_topk_attn_kernel, qi=qi),
            grid=(BH,),
            in_specs=[
                pl.BlockSpec((1, QB, D), lambda bh, qi=qi: (bh, qi, 0)),
                pl.BlockSpec((1, L, D), lambda bh: (bh, 0, 0)),
                pl.BlockSpec((1, L, D), lambda bh: (bh, 0, 0)),
            ],
            out_specs=pl.BlockSpec((1, QB, D), lambda bh: (bh, 0, 0)),
            out_shape=jax.ShapeDtypeStruct((BH, QB, D), jnp.float32),
            compiler_params=pltpu.CompilerParams(
                dimension_semantics=("arbitrary",),
            ),
        )(q3, k3, v3)
        outs.append(out_qi)
    out = jnp.concatenate(outs, axis=1)
    return out.reshape(B, H, T, D)


# speedup vs baseline: 44.3338x; 1.1085x over previous
import functools
import math

import jax
import jax.numpy as jnp
from jax.experimental import pallas as pl
from jax.experimental.pallas import tpu as pltpu

_K = 32
_QB = 256
_SEARCH_BITS = 24


def _topk_attn_kernel(q_ref, k_ref, v_ref, o_ref, *, qi):
    QB = q_ref.shape[1]
    L = k_ref.shape[1]
    D = q_ref.shape[2]
    scale = 1.0 / math.sqrt(D)

    q = q_ref[0]
    k = k_ref[0]
    s = jax.lax.dot_general(
        q, k, (((1,), (1,)), ((), ())), preferred_element_type=jnp.float32
    ) * scale

    row = qi * QB + jax.lax.broadcasted_iota(jnp.int32, (QB, L), 0)
    col = jax.lax.broadcasted_iota(jnp.int32, (QB, L), 1)
    s = jnp.where(col > row, -jnp.inf, s)

    m = jnp.max(s, axis=1, keepdims=True)
    w = s - m

    thr = jnp.full((QB, 1), -32.0, jnp.float32)
    step = 16.0
    for _ in range(_SEARCH_BITS):
        cand = thr + step
        cnt = jnp.sum((w >= cand).astype(jnp.int32), axis=1, keepdims=True)
        thr = jnp.where(cnt >= _K, cand, thr)
        step *= 0.5

    p = jnp.where(w >= thr, jnp.exp(w), 0.0)
    denom = jnp.sum(p, axis=1, keepdims=True)
    o = jax.lax.dot_general(
        p, v_ref[0], (((1,), (0,)), ((), ())), preferred_element_type=jnp.float32
    )
    o_ref[0] = o / denom


def kernel(q, k, v, k_sparse):
    B, H, T, D = q.shape
    QB = _QB
    BH = B * H
    NQ = T // QB
    q3 = q.reshape(BH, T, D)
    k3 = k.reshape(BH, T, D)
    v3 = v.reshape(BH, T, D)

    outs = []
    for qi in range(NQ):
        L = (qi + 1) * QB
        out_qi = pl.pallas_call(
            functools.partial(_topk_attn_kernel, qi=qi),
            grid=(BH,),
            in_specs=[
                pl.BlockSpec((1, QB, D), lambda bh, qi=qi: (bh, qi, 0)),
                pl.BlockSpec((1, L, D), lambda bh: (bh, 0, 0)),
                pl.BlockSpec((1, L, D), lambda bh: (bh, 0, 0)),
            ],
            out_specs=pl.BlockSpec((1, QB, D), lambda bh: (bh, 0, 0)),
            out_shape=jax.ShapeDtypeStruct((BH, QB, D), jnp.float32),
            compiler_params=pltpu.CompilerParams(
                dimension_semantics=("arbitrary",),
            ),
        )(q3, k3, v3)
        outs.append(out_qi)
    out = jnp.concatenate(outs, axis=1)
    return out.reshape(B, H, T, D)


# speedup vs baseline: 54.2123x; 1.2228x over previous
import functools
import math

import jax
import jax.numpy as jnp
from jax.experimental import pallas as pl
from jax.experimental.pallas import tpu as pltpu

_K = 32
_QB = 256
_SEARCH_BITS = 22


def _topk_attn_kernel(q_ref, k_ref, v_ref, o_ref, *, qi):
    QB = q_ref.shape[1]
    L = k_ref.shape[1]
    D = q_ref.shape[2]
    scale = 1.0 / math.sqrt(D)

    q = q_ref[0]
    k = k_ref[0]
    s = jax.lax.dot_general(
        q, k, (((1,), (1,)), ((), ())), preferred_element_type=jnp.float32
    ) * scale

    row = qi * QB + jax.lax.broadcasted_iota(jnp.int32, (QB, L), 0)
    col = jax.lax.broadcasted_iota(jnp.int32, (QB, L), 1)
    s = jnp.where(col > row, -jnp.inf, s)

    m = jnp.max(s, axis=1, keepdims=True)
    w = s - m

    thr = jnp.full((QB, 1), -8.0, jnp.float32)
    step = 4.0
    for _ in range(_SEARCH_BITS):
        cand = thr + step
        cnt = jnp.sum((w >= cand).astype(jnp.float32), axis=1, keepdims=True)
        thr = jnp.where(cnt >= float(_K), cand, thr)
        step *= 0.5

    p = jnp.where(w >= thr, jnp.exp(w), 0.0)
    denom = jnp.sum(p, axis=1, keepdims=True)
    o = jax.lax.dot_general(
        p, v_ref[0], (((1,), (0,)), ((), ())), preferred_element_type=jnp.float32
    )
    o_ref[0] = o / denom


def kernel(q, k, v, k_sparse):
    B, H, T, D = q.shape
    QB = _QB
    BH = B * H
    NQ = T // QB
    q3 = q.reshape(BH, T, D)
    k3 = k.reshape(BH, T, D)
    v3 = v.reshape(BH, T, D)

    outs = []
    for qi in range(NQ):
        L = (qi + 1) * QB
        out_qi = pl.pallas_call(
            functools.partial(_topk_attn_kernel, qi=qi),
            grid=(BH,),
            in_specs=[
                pl.BlockSpec((1, QB, D), lambda bh, qi=qi: (bh, qi, 0)),
                pl.BlockSpec((1, L, D), lambda bh: (bh, 0, 0)),
                pl.BlockSpec((1, L, D), lambda bh: (bh, 0, 0)),
            ],
            out_specs=pl.BlockSpec((1, QB, D), lambda bh: (bh, 0, 0)),
            out_shape=jax.ShapeDtypeStruct((BH, QB, D), jnp.float32),
            compiler_params=pltpu.CompilerParams(
                dimension_semantics=("arbitrary",),
            ),
        )(q3, k3, v3)
        outs.append(out_qi)
    out = jnp.concatenate(outs, axis=1)
    return out.reshape(B, H, T, D)
